# Initial kernel scaffold; baseline (speedup 1.0000x reference)
import jax
import jax.numpy as jnp
from jax.experimental import pallas as pl


def kernel(x, table):
    raise NotImplementedError("write your pallas kernel here")


# speedup vs baseline: 1.8564x; 1.8564x over previous
import functools

import jax
import jax.numpy as jnp
from jax import lax
from jax.experimental import pallas as pl
from jax.experimental.pallas import tpu as pltpu
from jax.experimental.pallas import tpu_sc as plsc

_B, _C, _H, _W = 8, 128, 224, 224
_HW = _H * _W
_V = 1000
_VP = 1024
_NC, _NS = 2, 16
_NW = _NC * _NS
_CBLK = 16
_NCB = _C // _CBLK
_BPW = _B * _NCB // _NW
_CH = 3584
_NCHUNK = _HW // _CH


def _transpose_table(tpad):
    def body(t_ref, o_ref):
        o_ref[...] = t_ref[...].T

    return pl.pallas_call(
        body, out_shape=jax.ShapeDtypeStruct((_C, _VP), jnp.float32)
    )(tpad)


def _sc_gather(table_t, idx):
    mesh = plsc.VectorSubcoreMesh(
        core_axis_name="c", subcore_axis_name="s",
        num_cores=_NC, num_subcores=_NS)

    @functools.partial(
        pl.kernel,
        out_type=jax.ShapeDtypeStruct((_B, _C, _HW), jnp.float32),
        mesh=mesh,
        compiler_params=pltpu.CompilerParams(needs_layout_passes=False),
        scratch_types=[
            pltpu.VMEM((_CBLK * _VP,), jnp.float32),
            pltpu.VMEM((_CH,), jnp.int32),
            pltpu.VMEM((_CBLK * _CH,), jnp.float32),
        ],
    )
    def k(tt_hbm, idx_hbm, out_hbm, lut_v, idx_v, stage_v):
        wid = lax.axis_index("s") * _NC + lax.axis_index("c")
        cblk = wid // (_NW // _NCB)
        bpair = wid % (_NW // _NCB)
        pltpu.sync_copy(tt_hbm.at[pl.ds(cblk * (_CBLK * _VP), _CBLK * _VP)],
                        lut_v)
        for j in range(_BPW):
            b = bpair * _BPW + j

            def chunk_body(ck, _):
                pltpu.sync_copy(idx_hbm.at[b, pl.ds(ck * _CH, _CH)], idx_v)

                def pos_body(i, _):
                    iv = idx_v[pl.ds(i * 16, 16)]
                    for c in range(_CBLK):
                        stage_v[pl.ds(c * _CH + i * 16, 16)] = plsc.load_gather(
                            lut_v, [iv + c * _VP])
                    return 0

                lax.fori_loop(0, _CH // 16, pos_body, 0)
                for c in range(_CBLK):
                    pltpu.sync_copy(
                        stage_v.at[pl.ds(c * _CH, _CH)],
                        out_hbm.at[b, cblk * _CBLK + c, pl.ds(ck * _CH, _CH)])
                return 0

            lax.fori_loop(0, _NCHUNK, chunk_body, 0)

    return k(table_t, idx)


def kernel(x, table):
    idx = x.reshape(_B, _HW)
    tpad = jnp.zeros((_VP, _C), jnp.float32).at[:_V].set(table)
    table_t = _transpose_table(tpad).reshape(_C * _VP)
    out = _sc_gather(table_t, idx)
    return out.reshape(_B, _C, _H, _W)
